# baseline (speedup 1.0000x reference)
import functools

import jax
import jax.numpy as jnp
from jax import lax
from jax.experimental import pallas as pl
from jax.experimental.pallas import tpu as pltpu
from jax.experimental.pallas import tpu_sc as plsc

N_NODES = 10000
N_EDGES = 320000
D_NODE = 128
D_EDGE = 16

NC = 2
NS = 16
NW = NC * NS

USUB = 128
UNSUB = 4
UEDGES = USUB * UNSUB
UNITS = N_EDGES // UEDGES
UITERS = -(-UNITS // NW)
IDXROWS = N_EDGES // USUB


def _node_proj_kernel(nf_ref, w_ref, p1_ref, p2_ref):
    nf = nf_ref[...]
    w1 = w_ref[0:D_NODE, :]
    w2 = w_ref[D_NODE:2 * D_NODE, :]
    p1_ref[...] = jnp.dot(nf, w1, preferred_element_type=jnp.float32)
    p2_ref[...] = jnp.dot(nf, w2, preferred_element_type=jnp.float32)


def _final_kernel(g_ref, ef_ref, w3_ref, b_ref, out_ref):
    e = lax.dot_general(w3_ref[...], ef_ref[...],
                        (((0,), (0,)), ((), ())),
                        preferred_element_type=jnp.float32)
    bb = jnp.broadcast_to(b_ref[...], e.shape)
    out_ref[...] = jnp.maximum(g_ref[...] + e + bb, 0.0)


def _sc_edge_kernel(p1_hbm, p2_hbm, idx_hbm, out_hbm,
                    sa, da, sb, db, g1a, g2a, g1b, g2b, oa, ob,
                    isa_, isb_, gsa, gsb, osa, osb):
    wid = lax.axis_index("s") * NC + lax.axis_index("c")
    lane = lax.iota(jnp.int32, D_EDGE)

    sets = ((sa, da, g1a, g2a, oa, isa_, gsa, osa),
            (sb, db, g1b, g2b, ob, isb_, gsb, osb))

    def uid(i):
        return jnp.minimum(wid * UITERS + i, UNITS - 1)

    def issue_idx(i, s):
        sv, dv, isem = s[0], s[1], s[5]
        u = uid(i)
        pltpu.async_copy(idx_hbm.at[0, pl.ds(u * UNSUB, UNSUB)], sv, isem)
        pltpu.async_copy(idx_hbm.at[1, pl.ds(u * UNSUB, UNSUB)], dv, isem)

    def wait_idx(s):
        sv, dv, isem = s[0], s[1], s[5]
        pltpu.make_async_copy(idx_hbm.at[0, pl.ds(0, UNSUB)], sv, isem).wait()
        pltpu.make_async_copy(idx_hbm.at[1, pl.ds(0, UNSUB)], dv, isem).wait()

    def issue_gathers(s):
        sv, dv, g1, g2, gsem = s[0], s[1], s[2], s[3], s[6]
        for j in range(UNSUB):
            pltpu.async_copy(p1_hbm.at[sv.at[j]],
                             g1.at[pl.ds(j * USUB, USUB)], gsem)
            pltpu.async_copy(p2_hbm.at[dv.at[j]],
                             g2.at[pl.ds(j * USUB, USUB)], gsem)

    def wait_gathers(s):
        sv, g1, g2, gsem = s[0], s[2], s[3], s[6]
        for j in range(UNSUB):
            pltpu.make_async_copy(p1_hbm.at[sv.at[0]],
                                  g1.at[pl.ds(j * USUB, USUB)], gsem).wait()
            pltpu.make_async_copy(p1_hbm.at[sv.at[0]],
                                  g2.at[pl.ds(j * USUB, USUB)], gsem).wait()

    def compute(s):
        g1, g2, o = s[2], s[3], s[4]
        zeros = lane * 0
        for t in range(UNSUB):
            d1 = zeros + t

            def row_body(i2):
                i = t * USUB + i2
                v = g1[i, :] + g2[i, :]
                plsc.store_scatter(o, [lane, d1, zeros + i2], v)

            plsc.parallel_loop(0, USUB, 1, unroll=8)(row_body)

    def issue_write(i, s):
        o, osem = s[4], s[7]
        u = uid(i)
        pltpu.async_copy(o, out_hbm.at[:, pl.ds(u * UNSUB, UNSUB), :], osem)

    def wait_write(s):
        o, osem = s[4], s[7]
        pltpu.make_async_copy(
            o, out_hbm.at[:, pl.ds(0, UNSUB), :], osem).wait()

    issue_idx(0, sets[0])
    wait_idx(sets[0])
    issue_gathers(sets[0])
    issue_idx(1, sets[1])
    for k in range(UITERS):
        cur = sets[k % 2]
        nxt = sets[(k + 1) % 2]
        if k + 1 < UITERS:
            wait_idx(nxt)
            issue_gathers(nxt)
        wait_gathers(cur)
        if k + 2 < UITERS:
            issue_idx(k + 2, cur)
        if k >= 2:
            wait_write(cur)
        compute(cur)
        issue_write(k, cur)
    wait_write(sets[(UITERS - 2) % 2])
    wait_write(sets[(UITERS - 1) % 2])


def kernel(node_feat, edge_index, edge_feat, W, b):
    p1, p2 = pl.pallas_call(
        _node_proj_kernel,
        grid=(10,),
        in_specs=[
            pl.BlockSpec((N_NODES // 10, D_NODE), lambda i: (i, 0)),
            pl.BlockSpec((2 * D_NODE, D_EDGE), lambda i: (0, 0)),
        ],
        out_specs=[
            pl.BlockSpec((N_NODES // 10, D_EDGE), lambda i: (i, 0)),
            pl.BlockSpec((N_NODES // 10, D_EDGE), lambda i: (i, 0)),
        ],
        out_shape=[
            jax.ShapeDtypeStruct((N_NODES, D_EDGE), jnp.float32),
            jax.ShapeDtypeStruct((N_NODES, D_EDGE), jnp.float32),
        ],
    )(node_feat, W[:2 * D_NODE])

    idx3d = edge_index.astype(jnp.int32).reshape(2, IDXROWS, USUB)
    mesh = plsc.VectorSubcoreMesh(
        core_axis_name="c", subcore_axis_name="s",
        num_cores=NC, num_subcores=NS)
    g3 = functools.partial(
        pl.kernel,
        out_type=jax.ShapeDtypeStruct((D_EDGE, IDXROWS, USUB), jnp.float32),
        mesh=mesh,
        scratch_types=[
            pltpu.VMEM((UNSUB, USUB), jnp.int32),
            pltpu.VMEM((UNSUB, USUB), jnp.int32),
            pltpu.VMEM((UNSUB, USUB), jnp.int32),
            pltpu.VMEM((UNSUB, USUB), jnp.int32),
            pltpu.VMEM((UEDGES, D_EDGE), jnp.float32),
            pltpu.VMEM((UEDGES, D_EDGE), jnp.float32),
            pltpu.VMEM((UEDGES, D_EDGE), jnp.float32),
            pltpu.VMEM((UEDGES, D_EDGE), jnp.float32),
            pltpu.VMEM((D_EDGE, UNSUB, USUB), jnp.float32),
            pltpu.VMEM((D_EDGE, UNSUB, USUB), jnp.float32),
            pltpu.SemaphoreType.DMA,
            pltpu.SemaphoreType.DMA,
            pltpu.SemaphoreType.DMA,
            pltpu.SemaphoreType.DMA,
            pltpu.SemaphoreType.DMA,
            pltpu.SemaphoreType.DMA,
        ],
        compiler_params=pltpu.CompilerParams(
            use_tc_tiling_on_sc=False, needs_layout_passes=False),
    )(_sc_edge_kernel)(p1, p2, idx3d)

    g_t = g3.reshape(D_EDGE, N_EDGES)
    ef_t = edge_feat.T
    NBLK = 25
    BLK = N_EDGES // NBLK
    out_t = pl.pallas_call(
        _final_kernel,
        grid=(NBLK,),
        in_specs=[
            pl.BlockSpec((D_EDGE, BLK), lambda i: (0, i)),
            pl.BlockSpec((D_EDGE, BLK), lambda i: (0, i)),
            pl.BlockSpec((D_EDGE, D_EDGE), lambda i: (0, 0)),
            pl.BlockSpec((D_EDGE, 1), lambda i: (0, 0)),
        ],
        out_specs=pl.BlockSpec((D_EDGE, BLK), lambda i: (0, i)),
        out_shape=jax.ShapeDtypeStruct((D_EDGE, N_EDGES), jnp.float32),
    )(g_t, ef_t, W[2 * D_NODE:], b.reshape(D_EDGE, 1))
    return out_t.T
